# baseline (speedup 1.0000x reference)
import functools

import jax
import jax.numpy as jnp
from jax import lax
from jax.experimental import pallas as pl
from jax.experimental.pallas import tpu as pltpu
from jax.experimental.pallas import tpu_sc as plsc

B = 32
NW = 2048
D = 1024
K = 256
CHUNK = 256
LANES = 16
NVR = NW // LANES


def _logits_body(feat_ref, wrong_ref, out_ref):
    w = wrong_ref[0]
    fcol = feat_ref[0].reshape(D, 1)
    res = jax.lax.dot_general(
        w, fcol,
        dimension_numbers=(((1,), (0,)), ((), ())),
        preferred_element_type=jnp.float32,
    )
    out_ref[...] = res.reshape(1, CHUNK, 1)


def _compute_logits(feat, wrong):
    grid = (B, NW // CHUNK)
    feat3 = feat.reshape(B, 1, D)
    out = pl.pallas_call(
        _logits_body,
        grid=grid,
        in_specs=[
            pl.BlockSpec((1, 1, D), lambda b, c: (b, 0, 0)),
            pl.BlockSpec((1, CHUNK, D), lambda b, c: (b, c, 0)),
        ],
        out_specs=pl.BlockSpec((1, CHUNK, 1), lambda b, c: (b, c, 0)),
        out_shape=jax.ShapeDtypeStruct((B, NW, 1), jnp.float32),
    )(feat3, wrong)
    return out.reshape(B, NW)


def _right_body(feat_ref, right_ref, out_ref):
    rb = jnp.broadcast_to(right_ref[0], (CHUNK, D))
    fcol = feat_ref[0].reshape(D, 1)
    res = jax.lax.dot_general(
        rb, fcol,
        dimension_numbers=(((1,), (0,)), ((), ())),
        preferred_element_type=jnp.float32,
    )
    out_ref[...] = res.reshape(1, CHUNK, 1)


def _compute_right_logit(feat, right):
    feat3 = feat.reshape(B, 1, D)
    right3 = right.reshape(B, 1, D)
    out = pl.pallas_call(
        _right_body,
        grid=(B,),
        in_specs=[
            pl.BlockSpec((1, 1, D), lambda b: (b, 0, 0)),
            pl.BlockSpec((1, 1, D), lambda b: (b, 0, 0)),
        ],
        out_specs=pl.BlockSpec((1, CHUNK, 1), lambda b: (b, 0, 0)),
        out_shape=jax.ShapeDtypeStruct((B, CHUNK, 1), jnp.float32),
    )(feat3, right3)
    return out[:, 0, 0].reshape(B, 1)


def _softmax_body(logits_ref, right_ref, probs_ref):
    l = logits_ref[...]
    rl = right_ref[...]
    m = jnp.maximum(jnp.max(l, axis=1, keepdims=True), rl)
    e = jnp.exp(l - m)
    er = jnp.exp(rl - m)
    z = jnp.sum(e, axis=1, keepdims=True) + er
    probs_ref[...] = e / z


def _compute_probs(logits, right_logit):
    return pl.pallas_call(
        _softmax_body,
        out_shape=jax.ShapeDtypeStruct((B, NW), jnp.float32),
    )(logits, right_logit)


_MESH = plsc.VectorSubcoreMesh(
    core_axis_name="c", subcore_axis_name="s", num_cores=2, num_subcores=16
)


def _topk_sc_body(probs_hbm, outp_hbm, outi_hbm,
                  row_v, kA, iA, kB, iB, hist, op_v, oi_v):
    wid = lax.axis_index("s") * 2 + lax.axis_index("c")
    pltpu.sync_copy(probs_hbm.at[wid], row_v)

    def init_body(v, carry):
        p = row_v[pl.ds(v * LANES, LANES)]
        bits = lax.bitcast_convert_type(p, jnp.int32)
        kA[pl.ds(v * LANES, LANES)] = ~bits
        iA[pl.ds(v * LANES, LANES)] = (
            v * LANES + lax.iota(jnp.int32, LANES))
        return carry
    lax.fori_loop(0, NVR, init_body, 0, unroll=2)

    bufs = [(kA, iA), (kB, iB)]
    for t in range(7):
        src_k, src_i = bufs[t % 2]
        dst_k, dst_i = bufs[(t + 1) % 2]
        shift = jnp.int32(5 * t)

        hist[pl.ds(0, LANES)] = jnp.zeros((LANES,), jnp.int32)
        hist[pl.ds(LANES, LANES)] = jnp.zeros((LANES,), jnp.int32)

        def hist_body(v, carry):
            kk = src_k[pl.ds(v * LANES, LANES)]
            d = lax.shift_right_logical(kk, shift) & jnp.int32(31)
            r1, last = plsc.scan_count(d)
            plsc.addupdate_scatter(hist, [d], r1, mask=last)
            return carry
        lax.fori_loop(0, NVR, hist_body, 0, unroll=2)

        h0 = hist[pl.ds(0, LANES)]
        h1 = hist[pl.ds(LANES, LANES)]
        b0 = plsc.cumsum(h0) - h0
        b1 = plsc.cumsum(h1) - h1 + jnp.sum(h0)
        hist[pl.ds(0, LANES)] = b0
        hist[pl.ds(LANES, LANES)] = b1

        def perm_body(v, carry):
            kk = src_k[pl.ds(v * LANES, LANES)]
            vv = src_i[pl.ds(v * LANES, LANES)]
            d = lax.shift_right_logical(kk, shift) & jnp.int32(31)
            r1, last = plsc.scan_count(d)
            base = plsc.load_gather(hist, [d])
            pos = base + r1 - 1
            plsc.store_scatter(dst_k, [pos], kk)
            plsc.store_scatter(dst_i, [pos], vv)
            plsc.store_scatter(hist, [d], base + r1, mask=last)
            return carry
        lax.fori_loop(0, NVR, perm_body, 0, unroll=2)

    fin_k, fin_i = bufs[1]
    def out_body(v, carry):
        kk = fin_k[pl.ds(v * LANES, LANES)]
        op_v[pl.ds(v * LANES, LANES)] = lax.bitcast_convert_type(
            ~kk, jnp.float32)
        oi_v[pl.ds(v * LANES, LANES)] = fin_i[pl.ds(v * LANES, LANES)]
        return carry
    lax.fori_loop(0, K // LANES, out_body, 0, unroll=2)

    pltpu.sync_copy(op_v, outp_hbm.at[wid])
    pltpu.sync_copy(oi_v, outi_hbm.at[wid])


def _topk_sc(probs):
    f = pl.kernel(
        _topk_sc_body,
        out_type=(
            jax.ShapeDtypeStruct((B, K), jnp.float32),
            jax.ShapeDtypeStruct((B, K), jnp.int32),
        ),
        mesh=_MESH,
        compiler_params=pltpu.CompilerParams(needs_layout_passes=False),
        scratch_types=[
            pltpu.VMEM((NW,), jnp.float32),
            pltpu.VMEM((NW,), jnp.int32),
            pltpu.VMEM((NW,), jnp.int32),
            pltpu.VMEM((NW,), jnp.int32),
            pltpu.VMEM((NW,), jnp.int32),
            pltpu.VMEM((32,), jnp.int32),
            pltpu.VMEM((K,), jnp.float32),
            pltpu.VMEM((K,), jnp.int32),
        ],
    )
    return f(probs)


def kernel(feat, right, wrong):
    logits = _compute_logits(feat, wrong)
    right_logit = _compute_right_logit(feat, right)
    probs = _compute_probs(logits, right_logit)
    sample_prob, sample_idx = _topk_sc(probs)
    return (sample_prob, sample_idx)


# speedup vs baseline: 1.5348x; 1.5348x over previous
import functools

import jax
import jax.numpy as jnp
from jax import lax
from jax.experimental import pallas as pl
from jax.experimental.pallas import tpu as pltpu
from jax.experimental.pallas import tpu_sc as plsc

B = 32
NW = 2048
D = 1024
K = 256
CHUNK = 1024
LANES = 16
NVR = NW // LANES


def _logits_body(feat_ref, wrong_ref, out_ref):
    w = wrong_ref[0]
    fcol = feat_ref[0].reshape(D, 1)
    res = jax.lax.dot_general(
        w, fcol,
        dimension_numbers=(((1,), (0,)), ((), ())),
        preferred_element_type=jnp.float32,
    )
    out_ref[...] = res.reshape(1, CHUNK, 1)


def _compute_logits(feat, wrong):
    grid = (B, NW // CHUNK)
    feat3 = feat.reshape(B, 1, D)
    out = pl.pallas_call(
        _logits_body,
        grid=grid,
        in_specs=[
            pl.BlockSpec((1, 1, D), lambda b, c: (b, 0, 0)),
            pl.BlockSpec((1, CHUNK, D), lambda b, c: (b, c, 0)),
        ],
        out_specs=pl.BlockSpec((1, CHUNK, 1), lambda b, c: (b, c, 0)),
        out_shape=jax.ShapeDtypeStruct((B, NW, 1), jnp.float32),
    )(feat3, wrong)
    return out.reshape(B, NW)


def _right_body(feat_ref, right_ref, out_ref):
    rb = jnp.broadcast_to(right_ref[0], (CHUNK, D))
    fcol = feat_ref[0].reshape(D, 1)
    res = jax.lax.dot_general(
        rb, fcol,
        dimension_numbers=(((1,), (0,)), ((), ())),
        preferred_element_type=jnp.float32,
    )
    out_ref[...] = res.reshape(1, CHUNK, 1)


def _compute_right_logit(feat, right):
    feat3 = feat.reshape(B, 1, D)
    right3 = right.reshape(B, 1, D)
    out = pl.pallas_call(
        _right_body,
        grid=(B,),
        in_specs=[
            pl.BlockSpec((1, 1, D), lambda b: (b, 0, 0)),
            pl.BlockSpec((1, 1, D), lambda b: (b, 0, 0)),
        ],
        out_specs=pl.BlockSpec((1, CHUNK, 1), lambda b: (b, 0, 0)),
        out_shape=jax.ShapeDtypeStruct((B, CHUNK, 1), jnp.float32),
    )(feat3, right3)
    return out[:, 0, 0].reshape(B, 1)


def _softmax_body(logits_ref, right_ref, probs_ref):
    l = logits_ref[...]
    rl = right_ref[...]
    m = jnp.maximum(jnp.max(l, axis=1, keepdims=True), rl)
    e = jnp.exp(l - m)
    er = jnp.exp(rl - m)
    z = jnp.sum(e, axis=1, keepdims=True) + er
    probs_ref[...] = e / z


def _compute_probs(logits, right_logit):
    return pl.pallas_call(
        _softmax_body,
        out_shape=jax.ShapeDtypeStruct((B, NW), jnp.float32),
    )(logits, right_logit)


_MESH = plsc.VectorSubcoreMesh(
    core_axis_name="c", subcore_axis_name="s", num_cores=2, num_subcores=16
)


def _topk_sc_body(probs_hbm, outp_hbm, outi_hbm,
                  row_v, kA, iA, kB, iB, hist, op_v, oi_v):
    wid = lax.axis_index("s") * 2 + lax.axis_index("c")
    pltpu.sync_copy(probs_hbm.at[wid], row_v)

    def init_body(v, carry):
        p = row_v[pl.ds(v * LANES, LANES)]
        bits = lax.bitcast_convert_type(p, jnp.int32)
        kA[pl.ds(v * LANES, LANES)] = ~bits
        iA[pl.ds(v * LANES, LANES)] = (
            v * LANES + lax.iota(jnp.int32, LANES))
        return carry
    lax.fori_loop(0, NVR, init_body, 0, unroll=2)

    bufs = [(kA, iA), (kB, iB)]
    for t in range(7):
        src_k, src_i = bufs[t % 2]
        dst_k, dst_i = bufs[(t + 1) % 2]
        shift = jnp.int32(5 * t)

        hist[pl.ds(0, LANES)] = jnp.zeros((LANES,), jnp.int32)
        hist[pl.ds(LANES, LANES)] = jnp.zeros((LANES,), jnp.int32)

        def hist_body(v, carry):
            kk = src_k[pl.ds(v * LANES, LANES)]
            d = lax.shift_right_logical(kk, shift) & jnp.int32(31)
            r1, last = plsc.scan_count(d)
            plsc.addupdate_scatter(hist, [d], r1, mask=last)
            return carry
        lax.fori_loop(0, NVR, hist_body, 0, unroll=2)

        h0 = hist[pl.ds(0, LANES)]
        h1 = hist[pl.ds(LANES, LANES)]
        b0 = plsc.cumsum(h0) - h0
        b1 = plsc.cumsum(h1) - h1 + jnp.sum(h0)
        hist[pl.ds(0, LANES)] = b0
        hist[pl.ds(LANES, LANES)] = b1

        def perm_body(v, carry):
            kk = src_k[pl.ds(v * LANES, LANES)]
            vv = src_i[pl.ds(v * LANES, LANES)]
            d = lax.shift_right_logical(kk, shift) & jnp.int32(31)
            r1, last = plsc.scan_count(d)
            base = plsc.load_gather(hist, [d])
            pos = base + r1 - 1
            plsc.store_scatter(dst_k, [pos], kk)
            plsc.store_scatter(dst_i, [pos], vv)
            plsc.store_scatter(hist, [d], base + r1, mask=last)
            return carry
        lax.fori_loop(0, NVR, perm_body, 0, unroll=2)

    fin_k, fin_i = bufs[1]
    def out_body(v, carry):
        kk = fin_k[pl.ds(v * LANES, LANES)]
        op_v[pl.ds(v * LANES, LANES)] = lax.bitcast_convert_type(
            ~kk, jnp.float32)
        oi_v[pl.ds(v * LANES, LANES)] = fin_i[pl.ds(v * LANES, LANES)]
        return carry
    lax.fori_loop(0, K // LANES, out_body, 0, unroll=2)

    pltpu.sync_copy(op_v, outp_hbm.at[wid])
    pltpu.sync_copy(oi_v, outi_hbm.at[wid])


def _topk_sc(probs):
    f = pl.kernel(
        _topk_sc_body,
        out_type=(
            jax.ShapeDtypeStruct((B, K), jnp.float32),
            jax.ShapeDtypeStruct((B, K), jnp.int32),
        ),
        mesh=_MESH,
        compiler_params=pltpu.CompilerParams(needs_layout_passes=False),
        scratch_types=[
            pltpu.VMEM((NW,), jnp.float32),
            pltpu.VMEM((NW,), jnp.int32),
            pltpu.VMEM((NW,), jnp.int32),
            pltpu.VMEM((NW,), jnp.int32),
            pltpu.VMEM((NW,), jnp.int32),
            pltpu.VMEM((32,), jnp.int32),
            pltpu.VMEM((K,), jnp.float32),
            pltpu.VMEM((K,), jnp.int32),
        ],
    )
    return f(probs)


def kernel(feat, right, wrong):
    logits = _compute_logits(feat, wrong)
    right_logit = _compute_right_logit(feat, right)
    probs = _compute_probs(logits, right_logit)
    sample_prob, sample_idx = _topk_sc(probs)
    return (sample_prob, sample_idx)


# speedup vs baseline: 1.5484x; 1.0089x over previous
import functools

import jax
import jax.numpy as jnp
from jax import lax
from jax.experimental import pallas as pl
from jax.experimental.pallas import tpu as pltpu
from jax.experimental.pallas import tpu_sc as plsc

B = 32
NW = 2048
D = 1024
K = 256
CHUNK = 2048
LANES = 16
NVR = NW // LANES


def _logits_body(feat_ref, wrong_ref, out_ref):
    w = wrong_ref[0]
    fcol = feat_ref[0].reshape(D, 1)
    res = jax.lax.dot_general(
        w, fcol,
        dimension_numbers=(((1,), (0,)), ((), ())),
        preferred_element_type=jnp.float32,
    )
    out_ref[...] = res.reshape(1, CHUNK, 1)


def _compute_logits(feat, wrong):
    grid = (B, NW // CHUNK)
    feat3 = feat.reshape(B, 1, D)
    out = pl.pallas_call(
        _logits_body,
        grid=grid,
        in_specs=[
            pl.BlockSpec((1, 1, D), lambda b, c: (b, 0, 0)),
            pl.BlockSpec((1, CHUNK, D), lambda b, c: (b, c, 0)),
        ],
        out_specs=pl.BlockSpec((1, CHUNK, 1), lambda b, c: (b, c, 0)),
        out_shape=jax.ShapeDtypeStruct((B, NW, 1), jnp.float32),
    )(feat3, wrong)
    return out.reshape(B, NW)


def _right_body(feat_ref, right_ref, out_ref):
    rb = jnp.broadcast_to(right_ref[0], (CHUNK, D))
    fcol = feat_ref[0].reshape(D, 1)
    res = jax.lax.dot_general(
        rb, fcol,
        dimension_numbers=(((1,), (0,)), ((), ())),
        preferred_element_type=jnp.float32,
    )
    out_ref[...] = res.reshape(1, CHUNK, 1)


def _compute_right_logit(feat, right):
    feat3 = feat.reshape(B, 1, D)
    right3 = right.reshape(B, 1, D)
    out = pl.pallas_call(
        _right_body,
        grid=(B,),
        in_specs=[
            pl.BlockSpec((1, 1, D), lambda b: (b, 0, 0)),
            pl.BlockSpec((1, 1, D), lambda b: (b, 0, 0)),
        ],
        out_specs=pl.BlockSpec((1, CHUNK, 1), lambda b: (b, 0, 0)),
        out_shape=jax.ShapeDtypeStruct((B, CHUNK, 1), jnp.float32),
    )(feat3, right3)
    return out[:, 0, 0].reshape(B, 1)


def _softmax_body(logits_ref, right_ref, probs_ref):
    l = logits_ref[...]
    rl = right_ref[...]
    m = jnp.maximum(jnp.max(l, axis=1, keepdims=True), rl)
    e = jnp.exp(l - m)
    er = jnp.exp(rl - m)
    z = jnp.sum(e, axis=1, keepdims=True) + er
    probs_ref[...] = e / z


def _compute_probs(logits, right_logit):
    return pl.pallas_call(
        _softmax_body,
        out_shape=jax.ShapeDtypeStruct((B, NW), jnp.float32),
    )(logits, right_logit)


_MESH = plsc.VectorSubcoreMesh(
    core_axis_name="c", subcore_axis_name="s", num_cores=2, num_subcores=16
)


def _topk_sc_body(probs_hbm, outp_hbm, outi_hbm,
                  row_v, kA, iA, kB, iB, hist, op_v, oi_v):
    wid = lax.axis_index("s") * 2 + lax.axis_index("c")
    pltpu.sync_copy(probs_hbm.at[wid], row_v)

    def init_body(v, carry):
        p = row_v[pl.ds(v * LANES, LANES)]
        bits = lax.bitcast_convert_type(p, jnp.int32)
        kA[pl.ds(v * LANES, LANES)] = ~bits
        iA[pl.ds(v * LANES, LANES)] = (
            v * LANES + lax.iota(jnp.int32, LANES))
        return carry
    lax.fori_loop(0, NVR, init_body, 0, unroll=2)

    bufs = [(kA, iA), (kB, iB)]
    for t in range(7):
        src_k, src_i = bufs[t % 2]
        dst_k, dst_i = bufs[(t + 1) % 2]
        shift = jnp.int32(5 * t)

        hist[pl.ds(0, LANES)] = jnp.zeros((LANES,), jnp.int32)
        hist[pl.ds(LANES, LANES)] = jnp.zeros((LANES,), jnp.int32)

        def hist_body(v, carry):
            kk = src_k[pl.ds(v * LANES, LANES)]
            d = lax.shift_right_logical(kk, shift) & jnp.int32(31)
            r1, last = plsc.scan_count(d)
            plsc.addupdate_scatter(hist, [d], r1, mask=last)
            return carry
        lax.fori_loop(0, NVR, hist_body, 0, unroll=2)

        h0 = hist[pl.ds(0, LANES)]
        h1 = hist[pl.ds(LANES, LANES)]
        b0 = plsc.cumsum(h0) - h0
        b1 = plsc.cumsum(h1) - h1 + jnp.sum(h0)
        hist[pl.ds(0, LANES)] = b0
        hist[pl.ds(LANES, LANES)] = b1

        def perm_body(v, carry):
            kk = src_k[pl.ds(v * LANES, LANES)]
            vv = src_i[pl.ds(v * LANES, LANES)]
            d = lax.shift_right_logical(kk, shift) & jnp.int32(31)
            r1, last = plsc.scan_count(d)
            base = plsc.load_gather(hist, [d])
            pos = base + r1 - 1
            plsc.store_scatter(dst_k, [pos], kk)
            plsc.store_scatter(dst_i, [pos], vv)
            plsc.store_scatter(hist, [d], base + r1, mask=last)
            return carry
        lax.fori_loop(0, NVR, perm_body, 0, unroll=2)

    fin_k, fin_i = bufs[1]
    def out_body(v, carry):
        kk = fin_k[pl.ds(v * LANES, LANES)]
        op_v[pl.ds(v * LANES, LANES)] = lax.bitcast_convert_type(
            ~kk, jnp.float32)
        oi_v[pl.ds(v * LANES, LANES)] = fin_i[pl.ds(v * LANES, LANES)]
        return carry
    lax.fori_loop(0, K // LANES, out_body, 0, unroll=2)

    pltpu.sync_copy(op_v, outp_hbm.at[wid])
    pltpu.sync_copy(oi_v, outi_hbm.at[wid])


def _topk_sc(probs):
    f = pl.kernel(
        _topk_sc_body,
        out_type=(
            jax.ShapeDtypeStruct((B, K), jnp.float32),
            jax.ShapeDtypeStruct((B, K), jnp.int32),
        ),
        mesh=_MESH,
        compiler_params=pltpu.CompilerParams(needs_layout_passes=False),
        scratch_types=[
            pltpu.VMEM((NW,), jnp.float32),
            pltpu.VMEM((NW,), jnp.int32),
            pltpu.VMEM((NW,), jnp.int32),
            pltpu.VMEM((NW,), jnp.int32),
            pltpu.VMEM((NW,), jnp.int32),
            pltpu.VMEM((32,), jnp.int32),
            pltpu.VMEM((K,), jnp.float32),
            pltpu.VMEM((K,), jnp.int32),
        ],
    )
    return f(probs)


def kernel(feat, right, wrong):
    logits = _compute_logits(feat, wrong)
    right_logit = _compute_right_logit(feat, right)
    probs = _compute_probs(logits, right_logit)
    sample_prob, sample_idx = _topk_sc(probs)
    return (sample_prob, sample_idx)
